# baseline (speedup 1.0000x reference)
import functools

import numpy as np
import jax
import jax.numpy as jnp
from jax import lax
from jax.experimental import pallas as pl
from jax.experimental.pallas import tpu as pltpu
from jax.experimental.pallas import tpu_sc as plsc

EMBEDDING_DIM = 1024
SEQUENCE_LEN = 160
VOCAB_SIZE = 100000
BATCH = 1024

NUM_CORES = 2
NUM_SUBCORES = 16
NW = NUM_CORES * NUM_SUBCORES
ROWS = BATCH * SEQUENCE_LEN
ROWS_PER_W = ROWS // NW
SEQS_PER_W = ROWS_PER_W // SEQUENCE_LEN
PBLK = 32
NPB = SEQUENCE_LEN // PBLK
LANES = 16
GROUPS = EMBEDDING_DIM // LANES
FACTOR = float(np.sqrt(EMBEDDING_DIM))


def _pos_encoding() -> np.ndarray:
    depth = EMBEDDING_DIM / 2
    positions = np.arange(SEQUENCE_LEN)[:, np.newaxis]
    depths = np.arange(depth)[np.newaxis, :] / depth
    rates = 1 / 10000 ** depths
    radians = positions * rates
    return np.concatenate(
        [np.sin(radians), np.cos(radians)], axis=-1).astype(np.float32)


_PE = _pos_encoding()


def _body(table, idx, pe, out, idx_v, pe_v, rows_v, sem):
    wid = lax.axis_index("s") * NUM_CORES + lax.axis_index("c")
    base = wid * ROWS_PER_W
    pltpu.sync_copy(idx.at[pl.ds(base, ROWS_PER_W)], idx_v)

    def pb_loop(pb, _):
        pltpu.sync_copy(pe.at[pl.ds(pb * PBLK, PBLK)], pe_v)

        def s_loop(s, _):
            off = s * SEQUENCE_LEN + pb * PBLK
            pltpu.async_copy(
                table.at[idx_v.at[pl.ds(off, PBLK)]], rows_v, sem).wait()

            def r_loop(r, _):
                def g_loop(g, _):
                    sl = pl.ds(g * LANES, LANES)
                    rows_v[r, sl] = rows_v[r, sl] * FACTOR + pe_v[r, sl]
                    return 0
                return lax.fori_loop(0, GROUPS, g_loop, 0)

            lax.fori_loop(0, PBLK, r_loop, 0)
            pltpu.sync_copy(rows_v, out.at[pl.ds(base + off, PBLK)])
            return 0

        lax.fori_loop(0, SEQS_PER_W, s_loop, 0)
        return 0

    lax.fori_loop(0, NPB, pb_loop, 0)


@jax.jit
def _embed(encoding, table):
    idx = encoding.reshape(ROWS).astype(jnp.int32)
    pe = jnp.asarray(_PE)
    mesh = plsc.VectorSubcoreMesh(core_axis_name="c", subcore_axis_name="s")
    k = pl.kernel(
        _body,
        out_type=jax.ShapeDtypeStruct((ROWS, EMBEDDING_DIM), jnp.float32),
        mesh=mesh,
        scratch_types=[
            pltpu.VMEM((ROWS_PER_W,), jnp.int32),
            pltpu.VMEM((PBLK, EMBEDDING_DIM), jnp.float32),
            pltpu.VMEM((PBLK, EMBEDDING_DIM), jnp.float32),
            pltpu.SemaphoreType.DMA,
        ],
    )
    out = k(table, idx, pe)
    return out.reshape(BATCH, SEQUENCE_LEN, EMBEDDING_DIM)


def kernel(encoding, table):
    return _embed(encoding, table)


# speedup vs baseline: 2.9635x; 2.9635x over previous
import numpy as np
import jax
import jax.numpy as jnp
from jax import lax
from jax.experimental import pallas as pl
from jax.experimental.pallas import tpu as pltpu
from jax.experimental.pallas import tpu_sc as plsc

EMBEDDING_DIM = 1024
SEQUENCE_LEN = 160
VOCAB_SIZE = 100000
BATCH = 1024

NUM_CORES = 2
NUM_SUBCORES = 16
NW = NUM_CORES * NUM_SUBCORES
ROWS = BATCH * SEQUENCE_LEN
ROWS_PER_W = ROWS // NW
SEQS_PER_W = ROWS_PER_W // SEQUENCE_LEN
CHUNK = 16
NCH = ROWS_PER_W // CHUNK
PBLK = 16
LANES = 16
GROUPS = EMBEDDING_DIM // LANES
FACTOR = float(np.sqrt(EMBEDDING_DIM))


def _pos_encoding() -> np.ndarray:
    depth = EMBEDDING_DIM / 2
    positions = np.arange(SEQUENCE_LEN)[:, np.newaxis]
    depths = np.arange(depth)[np.newaxis, :] / depth
    rates = 1 / 10000 ** depths
    radians = positions * rates
    return np.concatenate(
        [np.sin(radians), np.cos(radians)], axis=-1).astype(np.float32)


_PE = _pos_encoding()


def _chunk_off(c):
    return (c & 31) * SEQUENCE_LEN + (c >> 5) * PBLK


def _body(table, idx, pe, out, idx_v, pe_v, in0, in1, out0, out1,
          gs0, gs1, ws0, ws1):
    wid = lax.axis_index("s") * NUM_CORES + lax.axis_index("c")
    base = wid * ROWS_PER_W
    pltpu.sync_copy(idx.at[pl.ds(base, ROWS_PER_W)], idx_v)

    pltpu.async_copy(
        table.at[idx_v.at[pl.ds(_chunk_off(0), CHUNK)]], in0, gs0)
    pltpu.async_copy(
        table.at[idx_v.at[pl.ds(_chunk_off(1), CHUNK)]], in1, gs1)

    def step(c, inb, outb, gs, ws):
        off = _chunk_off(c)
        pltpu.make_async_copy(table.at[pl.ds(0, CHUNK)], inb, gs).wait()
        @pl.when(c >= 2)
        def _():
            pltpu.make_async_copy(outb, out.at[pl.ds(0, CHUNK)], ws).wait()

        def row(r, _):
            for g in range(GROUPS):
                sl = pl.ds(g * LANES, LANES)
                outb[r, sl] = inb[r, sl] * FACTOR + pe_v[r, sl]
            return 0
        lax.fori_loop(0, CHUNK, row, 0, unroll=2)

        pltpu.async_copy(outb, out.at[pl.ds(base + off, CHUNK)], ws)

        @pl.when(c < NCH - 2)
        def _():
            off2 = _chunk_off(c + 2)
            pltpu.async_copy(
                table.at[idx_v.at[pl.ds(off2, CHUNK)]], inb, gs)

    def c_loop(c, _):
        @pl.when((c & 31) == 0)
        def _():
            pltpu.sync_copy(pe.at[pl.ds((c >> 5) * PBLK, PBLK)], pe_v)

        @pl.when((c & 1) == 0)
        def _():
            step(c, in0, out0, gs0, ws0)

        @pl.when((c & 1) == 1)
        def _():
            step(c, in1, out1, gs1, ws1)
        return 0

    lax.fori_loop(0, NCH, c_loop, 0)

    pltpu.make_async_copy(out0, out.at[pl.ds(0, CHUNK)], ws0).wait()
    pltpu.make_async_copy(out1, out.at[pl.ds(0, CHUNK)], ws1).wait()


@jax.jit
def _embed(encoding, table):
    idx = encoding.reshape(ROWS).astype(jnp.int32)
    pe = jnp.asarray(_PE)
    mesh = plsc.VectorSubcoreMesh(core_axis_name="c", subcore_axis_name="s")
    k = pl.kernel(
        _body,
        out_type=jax.ShapeDtypeStruct((ROWS, EMBEDDING_DIM), jnp.float32),
        mesh=mesh,
        scratch_types=[
            pltpu.VMEM((ROWS_PER_W,), jnp.int32),
            pltpu.VMEM((PBLK, EMBEDDING_DIM), jnp.float32),
            pltpu.VMEM((CHUNK, EMBEDDING_DIM), jnp.float32),
            pltpu.VMEM((CHUNK, EMBEDDING_DIM), jnp.float32),
            pltpu.VMEM((CHUNK, EMBEDDING_DIM), jnp.float32),
            pltpu.VMEM((CHUNK, EMBEDDING_DIM), jnp.float32),
            pltpu.SemaphoreType.DMA,
            pltpu.SemaphoreType.DMA,
            pltpu.SemaphoreType.DMA,
            pltpu.SemaphoreType.DMA,
        ],
    )
    out = k(table, idx, pe)
    return out.reshape(BATCH, SEQUENCE_LEN, EMBEDDING_DIM)


def kernel(encoding, table):
    return _embed(encoding, table)
